# Initial kernel scaffold; baseline (speedup 1.0000x reference)
import jax
import jax.numpy as jnp
from jax.experimental import pallas as pl


def kernel(x_question, x_answer, W_lin_q, b_lin_q, W_lin_a, b_lin_a, Wk_qa, Wq_qa, Wv_qa, Wk_aq, Wq_aq, Wv_aq, ew_qa, ew_aq, edge_index_qa, edge_index_aq, edge_label_index):
    raise NotImplementedError("write your pallas kernel here")


# speedup vs baseline: 1.0472x; 1.0472x over previous
import functools
import numpy as np

import jax
import jax.numpy as jnp
from jax.experimental import pallas as pl
from jax.experimental.pallas import tpu as pltpu

N_NODE = 50000
D_IN = 128
H = 64
NH = 2
DH = 32
ROW_BLK = 5000
SQRT_DH = float(np.sqrt(DH))


def _proj_body(x_ref, wl_ref, b_ref, w1_ref, w2_ref, w3_ref,
               h_ref, a_ref, b2_ref, c_ref):
    x = x_ref[...]
    h = jnp.maximum(
        jnp.dot(x, wl_ref[...], preferred_element_type=jnp.float32)
        + b_ref[...][None, :], 0.0)
    h_ref[...] = h
    a_ref[...] = jnp.dot(h, w1_ref[...], preferred_element_type=jnp.float32)
    b2_ref[...] = jnp.dot(h, w2_ref[...], preferred_element_type=jnp.float32)
    c_ref[...] = jnp.dot(h, w3_ref[...], preferred_element_type=jnp.float32)


def _proj(x, wl, b, w1, w2, w3):
    n = x.shape[0]
    grid = (n // ROW_BLK,)
    out_sd = jax.ShapeDtypeStruct((n, H), jnp.float32)
    row_spec = pl.BlockSpec((ROW_BLK, D_IN), lambda i: (i, 0))
    out_spec = pl.BlockSpec((ROW_BLK, H), lambda i: (i, 0))
    full = lambda s: pl.BlockSpec(s, lambda i: tuple(0 for _ in s))
    return pl.pallas_call(
        _proj_body,
        grid=grid,
        in_specs=[row_spec, full((D_IN, H)), full((H,)),
                  full((H, H)), full((H, H)), full((H, H))],
        out_specs=[out_spec, out_spec, out_spec, out_spec],
        out_shape=[out_sd, out_sd, out_sd, out_sd],
    )(x, wl, b, w1, w2, w3)


def _edge_phase(K, Q, V, ei, ew, n_dst):
    src, dst = ei[0], ei[1]
    kk = K.reshape(-1, NH, DH)
    qq = Q.reshape(-1, NH, DH)
    vv = V.reshape(-1, NH, DH)
    alpha = (kk[src] * qq[dst]).sum(-1) / SQRT_DH
    ex = jnp.exp(alpha)
    den = jax.ops.segment_sum(ex, dst, num_segments=n_dst)
    msg = vv[src] * (ex * ew[:, None])[:, :, None]
    num = jax.ops.segment_sum(msg, dst, num_segments=n_dst)
    out = num / (den[:, :, None] + 1e-9)
    return out.reshape(n_dst, NH * DH)


def kernel(x_question, x_answer, W_lin_q, b_lin_q, W_lin_a, b_lin_a,
           Wk_qa, Wq_qa, Wv_qa, Wk_aq, Wq_aq, Wv_aq, ew_qa, ew_aq,
           edge_index_qa, edge_index_aq, edge_label_index):
    h_q, K_qa, V_qa, Q_aq = _proj(x_question, W_lin_q, b_lin_q,
                                  Wk_qa, Wv_qa, Wq_aq)
    h_a, K_aq, V_aq, Q_qa = _proj(x_answer, W_lin_a, b_lin_a,
                                  Wk_aq, Wv_aq, Wq_qa)
    m_a = _edge_phase(K_qa, Q_qa, V_qa, edge_index_qa, ew_qa, N_NODE)
    m_q = _edge_phase(K_aq, Q_aq, V_aq, edge_index_aq, ew_aq, N_NODE)
    z_q = jax.nn.relu(h_q + m_q)
    z_a = jax.nn.relu(h_a + m_a)
    s, d = edge_label_index[0], edge_label_index[1]
    pred = jax.nn.sigmoid((z_q[s] * z_a[d]).sum(-1))
    return pred
